# Initial kernel scaffold; baseline (speedup 1.0000x reference)
import jax
import jax.numpy as jnp
from jax.experimental import pallas as pl


def kernel(x, edge_index, Wself0, Wmsg0, bias0, Wself1, Wmsg1, bias1, Wself2, Wmsg2, bias2, Wsc1, bsc1, Wsc2, bsc2):
    raise NotImplementedError("write your pallas kernel here")


# speedup vs baseline: 2.7695x; 2.7695x over previous
import functools

import jax
import jax.numpy as jnp
from jax import lax
from jax.experimental import pallas as pl
from jax.experimental.pallas import tpu as pltpu
from jax.experimental.pallas import tpu_sc as plsc

F32 = jnp.float32
NC = 2
NS = 16
NW = NC * NS
LANES = 16
DH = 16
DW = 640

_SC_PARAMS = pltpu.CompilerParams(needs_layout_passes=False)


def _tc_in(x, wm, ws, b, n, d):
    def body(x_ref, wm_ref, ws_ref, b_ref, hm_ref, hs_ref):
        xb = x_ref[...]
        hm_ref[...] = jnp.dot(xb, wm_ref[...], preferred_element_type=F32)
        hs_ref[...] = jnp.dot(xb, ws_ref[...], preferred_element_type=F32) \
            + b_ref[...]
    return pl.pallas_call(
        body,
        out_shape=[jax.ShapeDtypeStruct((n, d), F32),
                   jax.ShapeDtypeStruct((n, d), F32)],
    )(x, wm, ws, b)


def _tc_combine(hs, aggp, inv, wm, ws, b, n, d):
    def body(hs_ref, a_ref, i_ref, wm_ref, ws_ref, b_ref, hm_ref, hso_ref):
        h = jnp.maximum(hs_ref[...] + (a_ref[0] + a_ref[1]) * i_ref[...], 0.0)
        hm_ref[...] = jnp.dot(h, wm_ref[...], preferred_element_type=F32)
        hso_ref[...] = jnp.dot(h, ws_ref[...], preferred_element_type=F32) \
            + b_ref[...]
    return pl.pallas_call(
        body,
        out_shape=[jax.ShapeDtypeStruct((n, d), F32),
                   jax.ShapeDtypeStruct((n, d), F32)],
    )(hs, aggp, inv, wm, ws, b)


def _sc_segsum(hm, src, dst, zrows, zdeg, with_deg, n, d, epw, c, nchunk,
               rm, tail):
    mesh = plsc.VectorSubcoreMesh(core_axis_name="c", subcore_axis_name="s")
    out_type = [jax.ShapeDtypeStruct((NC, n, d), F32)]
    if with_deg:
        out_type.append(jax.ShapeDtypeStruct((NW, DH, DW), F32))
    scratch = [
        pltpu.VMEM((c,), jnp.int32),
        pltpu.VMEM((c,), jnp.int32),
        pltpu.VMEM((c, d), F32),
        pltpu.VMEM((DH, DW), F32),
        pltpu.VMEM_SHARED((n, d), F32),
        pltpu.SemaphoreType.DMA,
    ]

    @functools.partial(pl.kernel, out_type=out_type, mesh=mesh,
                       compiler_params=_SC_PARAMS, scratch_types=scratch)
    def seg(hm_hbm, src_hbm, dst_hbm, zr_hbm, zd_hbm, *refs):
        if with_deg:
            agg_out, deg_out, src_v, dst_v, rows_v, deg_v, agg_s, sem = refs
        else:
            agg_out, src_v, dst_v, rows_v, deg_v, agg_s, sem = refs
        ci = lax.axis_index("c")
        si = lax.axis_index("s")
        wid = si * NC + ci
        one16 = jnp.full((LANES,), 1.0, F32)
        pltpu.sync_copy(zr_hbm, agg_s.at[pl.ds(si * rm, rm)])

        @pl.when(si == 0)
        def _():
            pltpu.sync_copy(zr_hbm.at[pl.ds(0, tail)],
                            agg_s.at[pl.ds(NS * rm, tail)])

        if with_deg:
            pltpu.sync_copy(zd_hbm, deg_v)
        plsc.subcore_barrier()

        def body(i, carry):
            base = wid * epw + i * c
            pltpu.sync_copy(dst_hbm.at[pl.ds(base, c)], dst_v)
            pltpu.sync_copy(src_hbm.at[pl.ds(base, c)], src_v)
            pltpu.async_copy(hm_hbm.at[src_v], rows_v, sem).wait()
            pltpu.sync_copy(rows_v, agg_s.at[dst_v], add=True)
            if with_deg:
                for j in range(c // LANES):
                    dvec = dst_v[pl.ds(j * LANES, LANES)]
                    plsc.addupdate_scatter(deg_v, [dvec // DW, dvec % DW],
                                           one16)
            return carry

        lax.fori_loop(0, nchunk, body, 0)
        if with_deg:
            pltpu.sync_copy(deg_v, deg_out.at[wid])
        plsc.subcore_barrier()
        pltpu.sync_copy(agg_s.at[pl.ds(si * rm, rm)],
                        agg_out.at[ci, pl.ds(si * rm, rm)])

        @pl.when(si == 0)
        def _():
            pltpu.sync_copy(agg_s.at[pl.ds(NS * rm, tail)],
                            agg_out.at[ci, pl.ds(NS * rm, tail)])

    return seg(hm, src, dst, zrows, zdeg)


def _sc_scorer(u, v, src, dst, w2, b2v, n, d, e, epw, c, nchunk):
    mesh = plsc.VectorSubcoreMesh(core_axis_name="c", subcore_axis_name="s")
    ngroups = c // LANES
    kin = d // LANES
    scratch = [
        pltpu.VMEM((c,), jnp.int32),
        pltpu.VMEM((c,), jnp.int32),
        pltpu.VMEM((c, d), F32),
        pltpu.VMEM((c, d), F32),
        pltpu.VMEM((c,), F32),
        pltpu.VMEM((d,), F32),
        pltpu.VMEM((LANES,), F32),
        pltpu.SemaphoreType.DMA,
    ]

    @functools.partial(
        pl.kernel, out_type=jax.ShapeDtypeStruct((e,), F32), mesh=mesh,
        compiler_params=_SC_PARAMS, scratch_types=scratch)
    def score(u_hbm, v_hbm, src_hbm, dst_hbm, w2_hbm, b2_hbm, out_hbm,
              src_v, dst_v, urows_v, vrows_v, out_v, w2_v, b2_v, sem):
        ci = lax.axis_index("c")
        si = lax.axis_index("s")
        wid = si * NC + ci
        pltpu.sync_copy(w2_hbm, w2_v)
        pltpu.sync_copy(b2_hbm, b2_v)
        lane = lax.iota(jnp.int32, LANES)

        def body(i, carry):
            base = wid * epw + i * c
            pltpu.sync_copy(src_hbm.at[pl.ds(base, c)], src_v)
            pltpu.sync_copy(dst_hbm.at[pl.ds(base, c)], dst_v)
            cu = pltpu.async_copy(u_hbm.at[src_v], urows_v, sem)
            cv = pltpu.async_copy(v_hbm.at[dst_v], vrows_v, sem)
            cu.wait()
            cv.wait()

            def group(g, carry2):
                eidx = lane + g * LANES
                acc0 = b2_v[...]

                def kstep(kb, acc):
                    w2seg = w2_v[pl.ds(kb * LANES, LANES)]
                    for kk in range(LANES):
                        kvec = jnp.full((LANES,), kk, jnp.int32) + kb * LANES
                        us = plsc.load_gather(urows_v, [eidx, kvec])
                        vs = plsc.load_gather(vrows_v, [eidx, kvec])
                        t = jnp.maximum(us + vs, 0.0)
                        acc = acc + t * w2seg[kk]
                    return acc

                acc0 = lax.fori_loop(0, kin, kstep, acc0)
                out_v[pl.ds(g * LANES, LANES)] = acc0
                return carry2

            lax.fori_loop(0, ngroups, group, 0)
            pltpu.sync_copy(out_v, out_hbm.at[pl.ds(base, c)])
            return carry

        lax.fori_loop(0, nchunk, body, 0)

    return score(u, v, src, dst, w2, b2v)


def kernel(x, edge_index, Wself0, Wmsg0, bias0, Wself1, Wmsg1, bias1,
           Wself2, Wmsg2, bias2, Wsc1, bsc1, Wsc2, bsc2):
    n, d = x.shape
    e = edge_index.shape[1]
    src = edge_index[0]
    dst = edge_index[1]
    epw = e // NW
    c = 80
    nchunk = epw // c
    rm = (n // (NS * 8)) * 8
    tail = n - NS * rm

    zrows = jnp.zeros((rm, d), F32)
    zdeg = jnp.zeros((DH, DW), F32)

    b0 = bias0.reshape(1, d)
    b1 = bias1.reshape(1, d)
    b2 = bias2.reshape(1, d)

    hm, hs = _tc_in(x, Wmsg0, Wself0, b0, n, d)
    aggp, degp = _sc_segsum(hm, src, dst, zrows, zdeg, True,
                            n, d, epw, c, nchunk, rm, tail)
    deg = degp.sum(axis=0).reshape(-1)[:n]
    inv = (1.0 / jnp.maximum(deg, 1.0)).reshape(n, 1)
    hm, hs = _tc_combine(hs, aggp, inv, Wmsg1, Wself1, b1, n, d)
    (aggp,) = _sc_segsum(hm, src, dst, zrows, zdeg, False,
                         n, d, epw, c, nchunk, rm, tail)
    hm, hs = _tc_combine(hs, aggp, inv, Wmsg2, Wself2, b2, n, d)
    (aggp,) = _sc_segsum(hm, src, dst, zrows, zdeg, False,
                         n, d, epw, c, nchunk, rm, tail)
    w1a = Wsc1[:d]
    w1b = Wsc1[d:]
    b1s = bsc1.reshape(1, d)
    v, u = _tc_combine(hs, aggp, inv, w1b, w1a, b1s, n, d)

    w2 = Wsc2[:, 0]
    b2v = jnp.broadcast_to(bsc2, (LANES,)).astype(F32)
    logits = _sc_scorer(u, v, src, dst, w2, b2v, n, d, e, epw, c, nchunk)
    return logits


# speedup vs baseline: 2.7702x; 1.0002x over previous
import functools

import jax
import jax.numpy as jnp
from jax import lax
from jax.experimental import pallas as pl
from jax.experimental.pallas import tpu as pltpu
from jax.experimental.pallas import tpu_sc as plsc

F32 = jnp.float32
NC = 2
NS = 16
NW = NC * NS
LANES = 16
DH = 16
DW = 640

_SC_PARAMS = pltpu.CompilerParams(needs_layout_passes=False)


def _tc_in(x, wm, ws, b, n, d):
    def body(x_ref, wm_ref, ws_ref, b_ref, hm_ref, hs_ref):
        xb = x_ref[...]
        hm_ref[...] = jnp.dot(xb, wm_ref[...], preferred_element_type=F32)
        hs_ref[...] = jnp.dot(xb, ws_ref[...], preferred_element_type=F32) \
            + b_ref[...]
    return pl.pallas_call(
        body,
        out_shape=[jax.ShapeDtypeStruct((n, d), F32),
                   jax.ShapeDtypeStruct((n, d), F32)],
    )(x, wm, ws, b)


def _tc_combine(hs, aggp, inv, wm, ws, b, n, d):
    def body(hs_ref, a_ref, i_ref, wm_ref, ws_ref, b_ref, hm_ref, hso_ref):
        h = jnp.maximum(hs_ref[...] + (a_ref[0] + a_ref[1]) * i_ref[...], 0.0)
        hm_ref[...] = jnp.dot(h, wm_ref[...], preferred_element_type=F32)
        hso_ref[...] = jnp.dot(h, ws_ref[...], preferred_element_type=F32) \
            + b_ref[...]
    return pl.pallas_call(
        body,
        out_shape=[jax.ShapeDtypeStruct((n, d), F32),
                   jax.ShapeDtypeStruct((n, d), F32)],
    )(hs, aggp, inv, wm, ws, b)


def _sc_segsum(hm, src, dst, zrows, zdeg, with_deg, n, d, epw, c, nchunk,
               rm, tail):
    mesh = plsc.VectorSubcoreMesh(core_axis_name="c", subcore_axis_name="s")
    out_type = [jax.ShapeDtypeStruct((NC, n, d), F32)]
    if with_deg:
        out_type.append(jax.ShapeDtypeStruct((NW, DH, DW), F32))
    scratch = [
        pltpu.VMEM((c,), jnp.int32),
        pltpu.VMEM((c,), jnp.int32),
        pltpu.VMEM((c, d), F32),
    ]
    if with_deg:
        scratch.append(pltpu.VMEM((DH, DW), F32))
    scratch += [
        pltpu.VMEM_SHARED((n, d), F32),
        pltpu.SemaphoreType.DMA,
    ]

    @functools.partial(pl.kernel, out_type=out_type, mesh=mesh,
                       compiler_params=_SC_PARAMS, scratch_types=scratch)
    def seg(hm_hbm, src_hbm, dst_hbm, zr_hbm, zd_hbm, *refs):
        if with_deg:
            agg_out, deg_out, src_v, dst_v, rows_v, deg_v, agg_s, sem = refs
        else:
            agg_out, src_v, dst_v, rows_v, agg_s, sem = refs
        ci = lax.axis_index("c")
        si = lax.axis_index("s")
        wid = si * NC + ci
        one16 = jnp.full((LANES,), 1.0, F32)
        pltpu.sync_copy(zr_hbm, agg_s.at[pl.ds(si * rm, rm)])

        @pl.when(si == 0)
        def _():
            pltpu.sync_copy(zr_hbm.at[pl.ds(0, tail)],
                            agg_s.at[pl.ds(NS * rm, tail)])

        if with_deg:
            pltpu.sync_copy(zd_hbm, deg_v)
        plsc.subcore_barrier()

        def body(i, carry):
            base = wid * epw + i * c
            pltpu.sync_copy(dst_hbm.at[pl.ds(base, c)], dst_v)
            pltpu.sync_copy(src_hbm.at[pl.ds(base, c)], src_v)
            pltpu.async_copy(hm_hbm.at[src_v], rows_v, sem).wait()
            pltpu.sync_copy(rows_v, agg_s.at[dst_v], add=True)
            if with_deg:
                for j in range(c // LANES):
                    dvec = dst_v[pl.ds(j * LANES, LANES)]
                    plsc.addupdate_scatter(deg_v, [dvec // DW, dvec % DW],
                                           one16)
            return carry

        lax.fori_loop(0, nchunk, body, 0)
        if with_deg:
            pltpu.sync_copy(deg_v, deg_out.at[wid])
        plsc.subcore_barrier()
        pltpu.sync_copy(agg_s.at[pl.ds(si * rm, rm)],
                        agg_out.at[ci, pl.ds(si * rm, rm)])

        @pl.when(si == 0)
        def _():
            pltpu.sync_copy(agg_s.at[pl.ds(NS * rm, tail)],
                            agg_out.at[ci, pl.ds(NS * rm, tail)])

    return seg(hm, src, dst, zrows, zdeg)


def _sc_scorer(u, v, src, dst, w2, b2v, n, d, e, epw, c, nchunk):
    mesh = plsc.VectorSubcoreMesh(core_axis_name="c", subcore_axis_name="s")
    ngroups = c // LANES
    kin = d // LANES
    scratch = [
        pltpu.VMEM((c,), jnp.int32),
        pltpu.VMEM((c,), jnp.int32),
        pltpu.VMEM((c, d), F32),
        pltpu.VMEM((c, d), F32),
        pltpu.VMEM((c,), F32),
        pltpu.VMEM((d,), F32),
        pltpu.VMEM((LANES,), F32),
        pltpu.SemaphoreType.DMA,
    ]

    @functools.partial(
        pl.kernel, out_type=jax.ShapeDtypeStruct((e,), F32), mesh=mesh,
        compiler_params=_SC_PARAMS, scratch_types=scratch)
    def score(u_hbm, v_hbm, src_hbm, dst_hbm, w2_hbm, b2_hbm, out_hbm,
              src_v, dst_v, urows_v, vrows_v, out_v, w2_v, b2_v, sem):
        ci = lax.axis_index("c")
        si = lax.axis_index("s")
        wid = si * NC + ci
        pltpu.sync_copy(w2_hbm, w2_v)
        pltpu.sync_copy(b2_hbm, b2_v)
        lane = lax.iota(jnp.int32, LANES)

        def body(i, carry):
            base = wid * epw + i * c
            pltpu.sync_copy(src_hbm.at[pl.ds(base, c)], src_v)
            pltpu.sync_copy(dst_hbm.at[pl.ds(base, c)], dst_v)
            cu = pltpu.async_copy(u_hbm.at[src_v], urows_v, sem)
            cv = pltpu.async_copy(v_hbm.at[dst_v], vrows_v, sem)
            cu.wait()
            cv.wait()

            def group(g, carry2):
                eidx = lane + g * LANES
                acc0 = b2_v[...]

                def kstep(kb, acc):
                    w2seg = w2_v[pl.ds(kb * LANES, LANES)]
                    for kk in range(LANES):
                        kvec = jnp.full((LANES,), kk, jnp.int32) + kb * LANES
                        us = plsc.load_gather(urows_v, [eidx, kvec])
                        vs = plsc.load_gather(vrows_v, [eidx, kvec])
                        t = jnp.maximum(us + vs, 0.0)
                        acc = acc + t * w2seg[kk]
                    return acc

                acc0 = lax.fori_loop(0, kin, kstep, acc0)
                out_v[pl.ds(g * LANES, LANES)] = acc0
                return carry2

            lax.fori_loop(0, ngroups, group, 0)
            pltpu.sync_copy(out_v, out_hbm.at[pl.ds(base, c)])
            return carry

        lax.fori_loop(0, nchunk, body, 0)

    return score(u, v, src, dst, w2, b2v)


def kernel(x, edge_index, Wself0, Wmsg0, bias0, Wself1, Wmsg1, bias1,
           Wself2, Wmsg2, bias2, Wsc1, bsc1, Wsc2, bsc2):
    n, d = x.shape
    e = edge_index.shape[1]
    src = edge_index[0]
    dst = edge_index[1]
    epw = e // NW
    c = 80
    nchunk = epw // c
    c2 = 80
    nchunk2 = epw // c2
    rm = (n // (NS * 8)) * 8
    tail = n - NS * rm

    zrows = jnp.zeros((rm, d), F32)
    zdeg = jnp.zeros((DH, DW), F32)

    b0 = bias0.reshape(1, d)
    b1 = bias1.reshape(1, d)
    b2 = bias2.reshape(1, d)

    hm, hs = _tc_in(x, Wmsg0, Wself0, b0, n, d)
    aggp, degp = _sc_segsum(hm, src, dst, zrows, zdeg, True,
                            n, d, epw, c, nchunk, rm, tail)
    deg = degp.sum(axis=0).reshape(-1)[:n]
    inv = (1.0 / jnp.maximum(deg, 1.0)).reshape(n, 1)
    hm, hs = _tc_combine(hs, aggp, inv, Wmsg1, Wself1, b1, n, d)
    (aggp,) = _sc_segsum(hm, src, dst, zrows, zdeg, False,
                         n, d, epw, c, nchunk, rm, tail)
    hm, hs = _tc_combine(hs, aggp, inv, Wmsg2, Wself2, b2, n, d)
    (aggp,) = _sc_segsum(hm, src, dst, zrows, zdeg, False,
                         n, d, epw, c, nchunk, rm, tail)
    w1a = Wsc1[:d]
    w1b = Wsc1[d:]
    b1s = bsc1.reshape(1, d)
    v, u = _tc_combine(hs, aggp, inv, w1b, w1a, b1s, n, d)

    w2 = Wsc2[:, 0]
    b2v = jnp.broadcast_to(bsc2, (LANES,)).astype(F32)
    logits = _sc_scorer(u, v, src, dst, w2, b2v, n, d, e, epw, c2, nchunk2)
    return logits


# speedup vs baseline: 2.9680x; 1.0714x over previous
import functools

import jax
import jax.numpy as jnp
from jax import lax
from jax.experimental import pallas as pl
from jax.experimental.pallas import tpu as pltpu
from jax.experimental.pallas import tpu_sc as plsc

F32 = jnp.float32
NC = 2
NS = 16
NW = NC * NS
LANES = 16
DH = 16
DW = 640

_SC_PARAMS = pltpu.CompilerParams(needs_layout_passes=False)


def _tc_in(x, wm, ws, b, n, d):
    def body(x_ref, wm_ref, ws_ref, b_ref, hm_ref, hs_ref):
        xb = x_ref[...]
        hm_ref[...] = jnp.dot(xb, wm_ref[...], preferred_element_type=F32)
        hs_ref[...] = jnp.dot(xb, ws_ref[...], preferred_element_type=F32) \
            + b_ref[...]
    return pl.pallas_call(
        body,
        out_shape=[jax.ShapeDtypeStruct((n, d), F32),
                   jax.ShapeDtypeStruct((n, d), F32)],
    )(x, wm, ws, b)


def _tc_combine(hs, aggp, inv, wm, ws, b, n, d):
    def body(hs_ref, a_ref, i_ref, wm_ref, ws_ref, b_ref, hm_ref, hso_ref):
        h = jnp.maximum(hs_ref[...] + (a_ref[0] + a_ref[1]) * i_ref[...], 0.0)
        hm_ref[...] = jnp.dot(h, wm_ref[...], preferred_element_type=F32)
        hso_ref[...] = jnp.dot(h, ws_ref[...], preferred_element_type=F32) \
            + b_ref[...]
    return pl.pallas_call(
        body,
        out_shape=[jax.ShapeDtypeStruct((n, d), F32),
                   jax.ShapeDtypeStruct((n, d), F32)],
    )(hs, aggp, inv, wm, ws, b)


def _sc_segsum(hm, src, dst, zrows, zdeg, with_deg, n, d, epw, c, nchunk,
               rm, tail):
    mesh = plsc.VectorSubcoreMesh(core_axis_name="c", subcore_axis_name="s")
    out_type = [jax.ShapeDtypeStruct((NC, n, d), F32)]
    if with_deg:
        out_type.append(jax.ShapeDtypeStruct((NW, DH, DW), F32))
    scratch = [
        pltpu.VMEM((c,), jnp.int32),
        pltpu.VMEM((c,), jnp.int32),
        pltpu.VMEM((c, d), F32),
    ]
    if with_deg:
        scratch.append(pltpu.VMEM((DH, DW), F32))
    scratch += [
        pltpu.VMEM_SHARED((n, d), F32),
        pltpu.SemaphoreType.DMA,
    ]

    @functools.partial(pl.kernel, out_type=out_type, mesh=mesh,
                       compiler_params=_SC_PARAMS, scratch_types=scratch)
    def seg(hm_hbm, src_hbm, dst_hbm, zr_hbm, zd_hbm, *refs):
        if with_deg:
            agg_out, deg_out, src_v, dst_v, rows_v, deg_v, agg_s, sem = refs
        else:
            agg_out, src_v, dst_v, rows_v, agg_s, sem = refs
        ci = lax.axis_index("c")
        si = lax.axis_index("s")
        wid = si * NC + ci
        one16 = jnp.full((LANES,), 1.0, F32)
        pltpu.sync_copy(zr_hbm, agg_s.at[pl.ds(si * rm, rm)])

        @pl.when(si == 0)
        def _():
            pltpu.sync_copy(zr_hbm.at[pl.ds(0, tail)],
                            agg_s.at[pl.ds(NS * rm, tail)])

        if with_deg:
            pltpu.sync_copy(zd_hbm, deg_v)
        plsc.subcore_barrier()

        def body(i, carry):
            base = wid * epw + i * c
            pltpu.sync_copy(dst_hbm.at[pl.ds(base, c)], dst_v)
            pltpu.sync_copy(src_hbm.at[pl.ds(base, c)], src_v)
            pltpu.async_copy(hm_hbm.at[src_v], rows_v, sem).wait()
            pltpu.sync_copy(rows_v, agg_s.at[dst_v], add=True)
            if with_deg:
                for j in range(c // LANES):
                    dvec = dst_v[pl.ds(j * LANES, LANES)]
                    plsc.addupdate_scatter(deg_v, [dvec // DW, dvec % DW],
                                           one16)
            return carry

        lax.fori_loop(0, nchunk, body, 0)
        if with_deg:
            pltpu.sync_copy(deg_v, deg_out.at[wid])
        plsc.subcore_barrier()
        pltpu.sync_copy(agg_s.at[pl.ds(si * rm, rm)],
                        agg_out.at[ci, pl.ds(si * rm, rm)])

        @pl.when(si == 0)
        def _():
            pltpu.sync_copy(agg_s.at[pl.ds(NS * rm, tail)],
                            agg_out.at[ci, pl.ds(NS * rm, tail)])

    return seg(hm, src, dst, zrows, zdeg)


def _sc_scorer(u, v, src, dst, w2, b2v, n, d, e, epw, c, nchunk):
    mesh = plsc.VectorSubcoreMesh(core_axis_name="c", subcore_axis_name="s")
    ngroups = c // LANES
    kin = d // LANES
    scratch = [
        pltpu.VMEM((c,), jnp.int32),
        pltpu.VMEM((c,), jnp.int32),
        pltpu.VMEM((c, d), F32),
        pltpu.VMEM((c, d), F32),
        pltpu.VMEM((c,), F32),
        pltpu.VMEM((d,), F32),
        pltpu.VMEM((LANES,), F32),
        pltpu.SemaphoreType.DMA,
    ]

    @functools.partial(
        pl.kernel, out_type=jax.ShapeDtypeStruct((e,), F32), mesh=mesh,
        compiler_params=_SC_PARAMS, scratch_types=scratch)
    def score(u_hbm, v_hbm, src_hbm, dst_hbm, w2_hbm, b2_hbm, out_hbm,
              src_v, dst_v, urows_v, vrows_v, out_v, w2_v, b2_v, sem):
        ci = lax.axis_index("c")
        si = lax.axis_index("s")
        wid = si * NC + ci
        pltpu.sync_copy(w2_hbm, w2_v)
        pltpu.sync_copy(b2_hbm, b2_v)
        lane = lax.iota(jnp.int32, LANES)

        def body(i, carry):
            base = wid * epw + i * c
            pltpu.sync_copy(src_hbm.at[pl.ds(base, c)], src_v)
            pltpu.sync_copy(dst_hbm.at[pl.ds(base, c)], dst_v)
            cu = pltpu.async_copy(u_hbm.at[src_v], urows_v, sem)
            cv = pltpu.async_copy(v_hbm.at[dst_v], vrows_v, sem)
            cu.wait()
            cv.wait()

            def group(g, carry2):
                eidx = lane + g * LANES
                zero = jnp.zeros((LANES,), F32)
                accs = (b2_v[...], zero, zero, zero)

                def kstep(kb, accs):
                    w2seg = w2_v[pl.ds(kb * LANES, LANES)]
                    kbase = jnp.full((LANES,), 0, jnp.int32) + kb * LANES
                    accs = list(accs)
                    for kk in range(LANES):
                        us = plsc.load_gather(urows_v, [eidx, kbase + kk])
                        vs = plsc.load_gather(vrows_v, [eidx, kbase + kk])
                        t = jnp.maximum(us + vs, 0.0)
                        accs[kk % 4] = accs[kk % 4] + t * w2seg[kk]
                    return tuple(accs)

                a0, a1, a2, a3 = lax.fori_loop(0, kin, kstep, accs)
                out_v[pl.ds(g * LANES, LANES)] = (a0 + a1) + (a2 + a3)
                return carry2

            lax.fori_loop(0, ngroups, group, 0)
            pltpu.sync_copy(out_v, out_hbm.at[pl.ds(base, c)])
            return carry

        lax.fori_loop(0, nchunk, body, 0)

    return score(u, v, src, dst, w2, b2v)


def kernel(x, edge_index, Wself0, Wmsg0, bias0, Wself1, Wmsg1, bias1,
           Wself2, Wmsg2, bias2, Wsc1, bsc1, Wsc2, bsc2):
    n, d = x.shape
    e = edge_index.shape[1]
    src = edge_index[0]
    dst = edge_index[1]
    epw = e // NW
    c = 80
    nchunk = epw // c
    c2 = 80
    nchunk2 = epw // c2
    rm = (n // (NS * 8)) * 8
    tail = n - NS * rm

    zrows = jnp.zeros((rm, d), F32)
    zdeg = jnp.zeros((DH, DW), F32)

    b0 = bias0.reshape(1, d)
    b1 = bias1.reshape(1, d)
    b2 = bias2.reshape(1, d)

    hm, hs = _tc_in(x, Wmsg0, Wself0, b0, n, d)
    aggp, degp = _sc_segsum(hm, src, dst, zrows, zdeg, True,
                            n, d, epw, c, nchunk, rm, tail)
    deg = degp.sum(axis=0).reshape(-1)[:n]
    inv = (1.0 / jnp.maximum(deg, 1.0)).reshape(n, 1)
    hm, hs = _tc_combine(hs, aggp, inv, Wmsg1, Wself1, b1, n, d)
    (aggp,) = _sc_segsum(hm, src, dst, zrows, zdeg, False,
                         n, d, epw, c, nchunk, rm, tail)
    hm, hs = _tc_combine(hs, aggp, inv, Wmsg2, Wself2, b2, n, d)
    (aggp,) = _sc_segsum(hm, src, dst, zrows, zdeg, False,
                         n, d, epw, c, nchunk, rm, tail)
    w1a = Wsc1[:d]
    w1b = Wsc1[d:]
    b1s = bsc1.reshape(1, d)
    v, u = _tc_combine(hs, aggp, inv, w1b, w1a, b1s, n, d)

    w2 = Wsc2[:, 0]
    b2v = jnp.broadcast_to(bsc2, (LANES,)).astype(F32)
    logits = _sc_scorer(u, v, src, dst, w2, b2v, n, d, e, epw, c2, nchunk2)
    return logits


# speedup vs baseline: 3.4949x; 1.1775x over previous
import functools

import jax
import jax.numpy as jnp
from jax import lax
from jax.experimental import pallas as pl
from jax.experimental.pallas import tpu as pltpu
from jax.experimental.pallas import tpu_sc as plsc

F32 = jnp.float32
NC = 2
NS = 16
NW = NC * NS
LANES = 16
DH = 16
DW = 640

_SC_PARAMS = pltpu.CompilerParams(needs_layout_passes=False)


def _tc_in(x, wm, ws, b, n, d):
    def body(x_ref, wm_ref, ws_ref, b_ref, hm_ref, hs_ref):
        xb = x_ref[...]
        hm_ref[...] = jnp.dot(xb, wm_ref[...], preferred_element_type=F32)
        hs_ref[...] = jnp.dot(xb, ws_ref[...], preferred_element_type=F32) \
            + b_ref[...]
    return pl.pallas_call(
        body,
        out_shape=[jax.ShapeDtypeStruct((n, d), F32),
                   jax.ShapeDtypeStruct((n, d), F32)],
    )(x, wm, ws, b)


def _tc_combine(hs, aggp, inv, wm, ws, b, n, d):
    def body(hs_ref, a_ref, i_ref, wm_ref, ws_ref, b_ref, hm_ref, hso_ref):
        h = jnp.maximum(hs_ref[...] + (a_ref[0] + a_ref[1]) * i_ref[...], 0.0)
        hm_ref[...] = jnp.dot(h, wm_ref[...], preferred_element_type=F32)
        hso_ref[...] = jnp.dot(h, ws_ref[...], preferred_element_type=F32) \
            + b_ref[...]
    return pl.pallas_call(
        body,
        out_shape=[jax.ShapeDtypeStruct((n, d), F32),
                   jax.ShapeDtypeStruct((n, d), F32)],
    )(hs, aggp, inv, wm, ws, b)


def _sc_segsum(hm, src, dst, zrows, zdeg, with_deg, n, d, epw, c, nchunk,
               rm, tail):
    mesh = plsc.VectorSubcoreMesh(core_axis_name="c", subcore_axis_name="s")
    out_type = [jax.ShapeDtypeStruct((NC, n, d), F32)]
    if with_deg:
        out_type.append(jax.ShapeDtypeStruct((NW, DH, DW), F32))
    scratch = [
        pltpu.VMEM((c,), jnp.int32),
        pltpu.VMEM((c,), jnp.int32),
        pltpu.VMEM((c, d), F32),
        pltpu.VMEM((c,), jnp.int32),
        pltpu.VMEM((c,), jnp.int32),
        pltpu.VMEM((c, d), F32),
    ]
    if with_deg:
        scratch.append(pltpu.VMEM((DH, DW), F32))
    scratch += [
        pltpu.VMEM_SHARED((n, d), F32),
        pltpu.SemaphoreType.DMA,
        pltpu.SemaphoreType.DMA,
    ]

    @functools.partial(pl.kernel, out_type=out_type, mesh=mesh,
                       compiler_params=_SC_PARAMS, scratch_types=scratch)
    def seg(hm_hbm, src_hbm, dst_hbm, zr_hbm, zd_hbm, *refs):
        if with_deg:
            (agg_out, deg_out, src_va, dst_va, rows_va, src_vb, dst_vb,
             rows_vb, deg_v, agg_s, sema, semb) = refs
        else:
            (agg_out, src_va, dst_va, rows_va, src_vb, dst_vb, rows_vb,
             agg_s, sema, semb) = refs
        ci = lax.axis_index("c")
        si = lax.axis_index("s")
        wid = si * NC + ci
        one16 = jnp.full((LANES,), 1.0, F32)
        pltpu.sync_copy(zr_hbm, agg_s.at[pl.ds(si * rm, rm)])

        @pl.when(si == 0)
        def _():
            pltpu.sync_copy(zr_hbm.at[pl.ds(0, tail)],
                            agg_s.at[pl.ds(NS * rm, tail)])

        if with_deg:
            pltpu.sync_copy(zd_hbm, deg_v)
        plsc.subcore_barrier()

        bufs = ((src_va, dst_va, rows_va, sema),
                (src_vb, dst_vb, rows_vb, semb))

        def start(i, buf):
            src_v, dst_v, rows_v, sem = buf
            base = wid * epw + i * c
            pltpu.sync_copy(dst_hbm.at[pl.ds(base, c)], dst_v)
            pltpu.sync_copy(src_hbm.at[pl.ds(base, c)], src_v)
            pltpu.async_copy(hm_hbm.at[src_v], rows_v, sem)

        def finish(buf):
            src_v, dst_v, rows_v, sem = buf
            pltpu.make_async_copy(hm_hbm.at[src_v], rows_v, sem).wait()
            pltpu.sync_copy(rows_v, agg_s.at[dst_v], add=True)
            if with_deg:
                for j in range(c // LANES):
                    dvec = dst_v[pl.ds(j * LANES, LANES)]
                    plsc.addupdate_scatter(deg_v, [dvec // DW, dvec % DW],
                                           one16)

        start(0, bufs[0])

        def body(i, carry):
            @pl.when(i % 2 == 0)
            def _():
                @pl.when(i + 1 < nchunk)
                def _():
                    start(i + 1, bufs[1])
                finish(bufs[0])

            @pl.when(i % 2 == 1)
            def _():
                @pl.when(i + 1 < nchunk)
                def _():
                    start(i + 1, bufs[0])
                finish(bufs[1])
            return carry

        lax.fori_loop(0, nchunk, body, 0)
        if with_deg:
            pltpu.sync_copy(deg_v, deg_out.at[wid])
        plsc.subcore_barrier()
        pltpu.sync_copy(agg_s.at[pl.ds(si * rm, rm)],
                        agg_out.at[ci, pl.ds(si * rm, rm)])

        @pl.when(si == 0)
        def _():
            pltpu.sync_copy(agg_s.at[pl.ds(NS * rm, tail)],
                            agg_out.at[ci, pl.ds(NS * rm, tail)])

    return seg(hm, src, dst, zrows, zdeg)


def _sc_scorer(u, v, src, dst, w2, b2v, n, d, e, epw, c, nchunk):
    mesh = plsc.VectorSubcoreMesh(core_axis_name="c", subcore_axis_name="s")
    ngroups = c // LANES
    kin = d // LANES
    scratch = [
        pltpu.VMEM((c,), jnp.int32),
        pltpu.VMEM((c,), jnp.int32),
        pltpu.VMEM((c, d), F32),
        pltpu.VMEM((c, d), F32),
        pltpu.VMEM((c,), F32),
        pltpu.VMEM((d,), F32),
        pltpu.VMEM((LANES,), F32),
        pltpu.SemaphoreType.DMA,
    ]

    @functools.partial(
        pl.kernel, out_type=jax.ShapeDtypeStruct((e,), F32), mesh=mesh,
        compiler_params=_SC_PARAMS, scratch_types=scratch)
    def score(u_hbm, v_hbm, src_hbm, dst_hbm, w2_hbm, b2_hbm, out_hbm,
              src_v, dst_v, urows_v, vrows_v, out_v, w2_v, b2_v, sem):
        ci = lax.axis_index("c")
        si = lax.axis_index("s")
        wid = si * NC + ci
        pltpu.sync_copy(w2_hbm, w2_v)
        pltpu.sync_copy(b2_hbm, b2_v)
        lane = lax.iota(jnp.int32, LANES)

        def body(i, carry):
            base = wid * epw + i * c
            pltpu.sync_copy(src_hbm.at[pl.ds(base, c)], src_v)
            pltpu.sync_copy(dst_hbm.at[pl.ds(base, c)], dst_v)
            cu = pltpu.async_copy(u_hbm.at[src_v], urows_v, sem)
            cv = pltpu.async_copy(v_hbm.at[dst_v], vrows_v, sem)
            cu.wait()
            cv.wait()

            def group(g, carry2):
                eidx = lane + g * LANES
                zero = jnp.zeros((LANES,), F32)
                accs = (b2_v[...], zero, zero, zero)

                def kstep(kb, accs):
                    w2seg = w2_v[pl.ds(kb * LANES, LANES)]
                    kbase = jnp.full((LANES,), 0, jnp.int32) + kb * LANES
                    accs = list(accs)
                    for kk in range(LANES):
                        us = plsc.load_gather(urows_v, [eidx, kbase + kk])
                        vs = plsc.load_gather(vrows_v, [eidx, kbase + kk])
                        t = jnp.maximum(us + vs, 0.0)
                        accs[kk % 4] = accs[kk % 4] + t * w2seg[kk]
                    return tuple(accs)

                a0, a1, a2, a3 = lax.fori_loop(0, kin, kstep, accs)
                out_v[pl.ds(g * LANES, LANES)] = (a0 + a1) + (a2 + a3)
                return carry2

            lax.fori_loop(0, ngroups, group, 0)
            pltpu.sync_copy(out_v, out_hbm.at[pl.ds(base, c)])
            return carry

        lax.fori_loop(0, nchunk, body, 0)

    return score(u, v, src, dst, w2, b2v)


def kernel(x, edge_index, Wself0, Wmsg0, bias0, Wself1, Wmsg1, bias1,
           Wself2, Wmsg2, bias2, Wsc1, bsc1, Wsc2, bsc2):
    n, d = x.shape
    e = edge_index.shape[1]
    src = edge_index[0]
    dst = edge_index[1]
    epw = e // NW
    c = 80
    nchunk = epw // c
    c2 = 80
    nchunk2 = epw // c2
    rm = (n // (NS * 8)) * 8
    tail = n - NS * rm

    zrows = jnp.zeros((rm, d), F32)
    zdeg = jnp.zeros((DH, DW), F32)

    b0 = bias0.reshape(1, d)
    b1 = bias1.reshape(1, d)
    b2 = bias2.reshape(1, d)

    hm, hs = _tc_in(x, Wmsg0, Wself0, b0, n, d)
    aggp, degp = _sc_segsum(hm, src, dst, zrows, zdeg, True,
                            n, d, epw, c, nchunk, rm, tail)
    deg = degp.sum(axis=0).reshape(-1)[:n]
    inv = (1.0 / jnp.maximum(deg, 1.0)).reshape(n, 1)
    hm, hs = _tc_combine(hs, aggp, inv, Wmsg1, Wself1, b1, n, d)
    (aggp,) = _sc_segsum(hm, src, dst, zrows, zdeg, False,
                         n, d, epw, c, nchunk, rm, tail)
    hm, hs = _tc_combine(hs, aggp, inv, Wmsg2, Wself2, b2, n, d)
    (aggp,) = _sc_segsum(hm, src, dst, zrows, zdeg, False,
                         n, d, epw, c, nchunk, rm, tail)
    w1a = Wsc1[:d]
    w1b = Wsc1[d:]
    b1s = bsc1.reshape(1, d)
    v, u = _tc_combine(hs, aggp, inv, w1b, w1a, b1s, n, d)

    w2 = Wsc2[:, 0]
    b2v = jnp.broadcast_to(bsc2, (LANES,)).astype(F32)
    logits = _sc_scorer(u, v, src, dst, w2, b2v, n, d, e, epw, c2, nchunk2)
    return logits
